# baseline (speedup 1.0000x reference)
import functools

import jax
import jax.numpy as jnp
from jax import lax
from jax.experimental import pallas as pl
from jax.experimental.pallas import tpu as pltpu
from jax.experimental.pallas import tpu_sc as plsc

B = 16384
DU = 64
DG = 32
NC = 2
NS = 16
NW = NC * NS
BPW = B // NW
CHK = 128
NCHK = BPW // CHK


def _sc_gather_body(uidx_hbm, iidx_hbm, ut_hbm, it_hbm,
                    uo_hbm, io_hbm,
                    uidx_v, iidx_v, ub0, ub1, ib0, ib1, gsem, osem):
    wid = lax.axis_index("s") * NC + lax.axis_index("c")
    base = wid * BPW
    pltpu.sync_copy(uidx_hbm.at[pl.ds(base, BPW)], uidx_v)
    pltpu.sync_copy(iidx_hbm.at[pl.ds(base, BPW)], iidx_v)

    ubufs = (ub0, ub1)
    ibufs = (ib0, ib1)
    pending = [None, None]

    for c in range(NCHK):
        slot = c % 2
        ubuf, ibuf = ubufs[slot], ibufs[slot]
        if pending[slot] is not None:
            for wb in pending[slot]:
                wb.wait()
            pending[slot] = None

        def enqueue(b, _):
            uv = uidx_v[pl.ds(c * CHK + b * 16, 16)]
            iv = iidx_v[pl.ds(c * CHK + b * 16, 16)]
            for k in range(16):
                pltpu.make_async_copy(
                    ut_hbm.at[pl.ds(uv[k], 1)],
                    ubuf.at[pl.ds(b * 16 + k, 1)], gsem).start()
                pltpu.make_async_copy(
                    it_hbm.at[pl.ds(iv[k], 1)],
                    ibuf.at[pl.ds(b * 16 + k, 1)], gsem).start()
            return _

        lax.fori_loop(0, CHK // 16, enqueue, 0)

        def drain(j, _):
            pltpu.make_async_copy(
                ut_hbm.at[pl.ds(0, 1)], ubuf.at[pl.ds(0, 1)], gsem).wait()
            pltpu.make_async_copy(
                it_hbm.at[pl.ds(0, 1)], ibuf.at[pl.ds(0, 1)], gsem).wait()
            return _

        lax.fori_loop(0, CHK, drain, 0)

        obase = base + c * CHK
        uwb = pltpu.make_async_copy(ubuf, uo_hbm.at[pl.ds(obase, CHK)], osem)
        iwb = pltpu.make_async_copy(ibuf, io_hbm.at[pl.ds(obase, CHK)], osem)
        uwb.start()
        iwb.start()
        pending[slot] = (uwb, iwb)

    for p in pending:
        if p is not None:
            for wb in p:
                wb.wait()


@functools.cache
def _sc_gather():
    mesh = plsc.VectorSubcoreMesh(core_axis_name="c", subcore_axis_name="s",
                                  num_cores=NC, num_subcores=NS)
    return pl.kernel(
        _sc_gather_body,
        out_type=(
            jax.ShapeDtypeStruct((B, DU), jnp.float32),
            jax.ShapeDtypeStruct((B, DU), jnp.float32),
        ),
        mesh=mesh,
        scratch_types=[
            pltpu.VMEM((BPW,), jnp.int32),
            pltpu.VMEM((BPW,), jnp.int32),
            pltpu.VMEM((CHK, DU), jnp.float32),
            pltpu.VMEM((CHK, DU), jnp.float32),
            pltpu.VMEM((CHK, DU), jnp.float32),
            pltpu.VMEM((CHK, DU), jnp.float32),
            pltpu.SemaphoreType.DMA,
            pltpu.SemaphoreType.DMA,
        ],
        compiler_params=pltpu.CompilerParams(use_tc_tiling_on_sc=True),
    )


TRB = 2048


def _tr_body(x_ref, o_ref):
    o_ref[...] = x_ref[...].T


def _transpose_table(tbl_t, n_rows):
    grid = (pl.cdiv(n_rows, TRB),)
    return pl.pallas_call(
        _tr_body,
        grid=grid,
        in_specs=[pl.BlockSpec((DU, TRB), lambda n: (0, n))],
        out_specs=pl.BlockSpec((TRB, DU), lambda n: (n, 0)),
        out_shape=jax.ShapeDtypeStruct((n_rows, DU), jnp.float32),
    )(tbl_t)


BLK = 1024


def _mlp_body(u_ref, i_ref, gidx_ref, w1u_ref, w1i_ref, gt_ref, w1g_ref,
              b1_ref, w2_ref, b2_ref, w3_ref, b3_ref, w4_ref, b4_ref, o_ref):
    g_eff = gt_ref[...] @ w1g_ref[...] + b1_ref[...]
    gsel = jnp.where(gidx_ref[...] == 0, g_eff[0:1, :], g_eff[1:2, :])
    h = u_ref[...] @ w1u_ref[...] + i_ref[...] @ w1i_ref[...] + gsel
    h = jnp.maximum(h, 0.0)
    h = jnp.maximum(h @ w2_ref[...] + b2_ref[...], 0.0)
    h = jnp.maximum(h @ w3_ref[...] + b3_ref[...], 0.0)
    o_ref[...] = jnp.sum(h * w4_ref[...], axis=1) + b4_ref[0, 0]


def _mlp(u, i, gidx, w1u, w1i, gt, w1g, b1, w2, b2, w3, b3, w4row, b4):
    grid = (B // BLK,)
    full = lambda shape: pl.BlockSpec(shape, lambda n: (0, 0))
    return pl.pallas_call(
        _mlp_body,
        grid=grid,
        in_specs=[
            pl.BlockSpec((BLK, DU), lambda n: (n, 0)),
            pl.BlockSpec((BLK, DU), lambda n: (n, 0)),
            pl.BlockSpec((BLK, 1), lambda n: (n, 0)),
            full((DU, 128)),
            full((DU, 128)),
            full((2, DG)),
            full((DG, 128)),
            full((1, 128)),
            full((128, 64)),
            full((1, 64)),
            full((64, 32)),
            full((1, 32)),
            full((1, 32)),
            full((1, 1)),
        ],
        out_specs=pl.BlockSpec((BLK,), lambda n: (n,)),
        out_shape=jax.ShapeDtypeStruct((B,), jnp.float32),
    )(u, i, gidx, w1u, w1i, gt, w1g, b1, w2, b2, w3, b3, w4row, b4)


def kernel(user_idx, item_idx, gender_idx, user_table, item_table,
           gender_table, W1, b1, W2, b2, W3, b3, W4, b4):
    uidx = user_idx.astype(jnp.int32)
    iidx = item_idx.astype(jnp.int32)
    gidx = gender_idx.astype(jnp.int32).reshape(B, 1)
    ut = _transpose_table(user_table.T, user_table.shape[0])
    it = _transpose_table(item_table.T, item_table.shape[0])
    u, i = _sc_gather()(uidx, iidx, ut, it)
    out = _mlp(
        u, i, gidx,
        W1[:DU], W1[DU:2 * DU], gender_table, W1[2 * DU:], b1.reshape(1, 128),
        W2, b2.reshape(1, 64), W3, b3.reshape(1, 32),
        W4.reshape(1, DG), b4.reshape(1, 1),
    )
    return out


# speedup vs baseline: 1.3274x; 1.3274x over previous
import functools

import jax
import jax.numpy as jnp
from jax import lax
from jax.experimental import pallas as pl
from jax.experimental.pallas import tpu as pltpu
from jax.experimental.pallas import tpu_sc as plsc

B = 16384
DU = 64
DG = 32
NC = 2
NS = 16
NW = NC * NS
BPW = B // NW
CHK = 128
NCHK = BPW // CHK


def _sc_gather_body(uidx_hbm, iidx_hbm, ut_hbm, it_hbm,
                    uo_hbm, io_hbm,
                    uidx_v, iidx_v, ub0, ub1, ib0, ib1, gsem, osem):
    wid = lax.axis_index("s") * NC + lax.axis_index("c")
    base = wid * BPW
    pltpu.sync_copy(uidx_hbm.at[pl.ds(base, BPW)], uidx_v)
    pltpu.sync_copy(iidx_hbm.at[pl.ds(base, BPW)], iidx_v)

    ubufs = (ub0, ub1)
    ibufs = (ib0, ib1)
    pending = [None, None]

    for c in range(NCHK):
        slot = c % 2
        ubuf, ibuf = ubufs[slot], ibufs[slot]
        if pending[slot] is not None:
            for wb in pending[slot]:
                wb.wait()
            pending[slot] = None

        def enqueue(b, _):
            uv = uidx_v[pl.ds(c * CHK + b * 16, 16)]
            iv = iidx_v[pl.ds(c * CHK + b * 16, 16)]
            for k in range(16):
                pltpu.make_async_copy(
                    ut_hbm.at[pl.ds(uv[k], 1)],
                    ubuf.at[pl.ds(b * 16 + k, 1)], gsem).start()
                pltpu.make_async_copy(
                    it_hbm.at[pl.ds(iv[k], 1)],
                    ibuf.at[pl.ds(b * 16 + k, 1)], gsem).start()
            return _

        lax.fori_loop(0, CHK // 16, enqueue, 0)

        def drain(j, _):
            pltpu.make_async_copy(
                ut_hbm.at[pl.ds(0, 1)], ubuf.at[pl.ds(0, 1)], gsem).wait()
            pltpu.make_async_copy(
                it_hbm.at[pl.ds(0, 1)], ibuf.at[pl.ds(0, 1)], gsem).wait()
            return _

        lax.fori_loop(0, CHK, drain, 0)

        obase = base + c * CHK
        uwb = pltpu.make_async_copy(ubuf, uo_hbm.at[pl.ds(obase, CHK)], osem)
        iwb = pltpu.make_async_copy(ibuf, io_hbm.at[pl.ds(obase, CHK)], osem)
        uwb.start()
        iwb.start()
        pending[slot] = (uwb, iwb)

    for p in pending:
        if p is not None:
            for wb in p:
                wb.wait()


@functools.cache
def _sc_gather():
    mesh = plsc.VectorSubcoreMesh(core_axis_name="c", subcore_axis_name="s",
                                  num_cores=NC, num_subcores=NS)
    return pl.kernel(
        _sc_gather_body,
        out_type=(
            jax.ShapeDtypeStruct((B, DU), jnp.float32),
            jax.ShapeDtypeStruct((B, DU), jnp.float32),
        ),
        mesh=mesh,
        scratch_types=[
            pltpu.VMEM((BPW,), jnp.int32),
            pltpu.VMEM((BPW,), jnp.int32),
            pltpu.VMEM((CHK, DU), jnp.float32),
            pltpu.VMEM((CHK, DU), jnp.float32),
            pltpu.VMEM((CHK, DU), jnp.float32),
            pltpu.VMEM((CHK, DU), jnp.float32),
            pltpu.SemaphoreType.DMA,
            pltpu.SemaphoreType.DMA,
        ],
        compiler_params=pltpu.CompilerParams(use_tc_tiling_on_sc=True),
    )


BLK = 1024


def _mlp_body(u_ref, i_ref, gidx_ref, w1u_ref, w1i_ref, gt_ref, w1g_ref,
              b1_ref, w2_ref, b2_ref, w3_ref, b3_ref, w4_ref, b4_ref, o_ref):
    g_eff = gt_ref[...] @ w1g_ref[...] + b1_ref[...]
    gsel = jnp.where(gidx_ref[...] == 0, g_eff[0:1, :], g_eff[1:2, :])
    h = u_ref[...] @ w1u_ref[...] + i_ref[...] @ w1i_ref[...] + gsel
    h = jnp.maximum(h, 0.0)
    h = jnp.maximum(h @ w2_ref[...] + b2_ref[...], 0.0)
    h = jnp.maximum(h @ w3_ref[...] + b3_ref[...], 0.0)
    o_ref[...] = jnp.sum(h * w4_ref[...], axis=1) + b4_ref[0, 0]


def _mlp(u, i, gidx, w1u, w1i, gt, w1g, b1, w2, b2, w3, b3, w4row, b4):
    grid = (B // BLK,)
    full = lambda shape: pl.BlockSpec(shape, lambda n: (0, 0))
    return pl.pallas_call(
        _mlp_body,
        grid=grid,
        in_specs=[
            pl.BlockSpec((BLK, DU), lambda n: (n, 0)),
            pl.BlockSpec((BLK, DU), lambda n: (n, 0)),
            pl.BlockSpec((BLK, 1), lambda n: (n, 0)),
            full((DU, 128)),
            full((DU, 128)),
            full((2, DG)),
            full((DG, 128)),
            full((1, 128)),
            full((128, 64)),
            full((1, 64)),
            full((64, 32)),
            full((1, 32)),
            full((1, 32)),
            full((1, 1)),
        ],
        out_specs=pl.BlockSpec((BLK,), lambda n: (n,)),
        out_shape=jax.ShapeDtypeStruct((B,), jnp.float32),
    )(u, i, gidx, w1u, w1i, gt, w1g, b1, w2, b2, w3, b3, w4row, b4)


def kernel(user_idx, item_idx, gender_idx, user_table, item_table,
           gender_table, W1, b1, W2, b2, W3, b3, W4, b4):
    uidx = user_idx.astype(jnp.int32)
    iidx = item_idx.astype(jnp.int32)
    gidx = gender_idx.astype(jnp.int32).reshape(B, 1)
    u, i = _sc_gather()(uidx, iidx, user_table, item_table)
    out = _mlp(
        u, i, gidx,
        W1[:DU], W1[DU:2 * DU], gender_table, W1[2 * DU:], b1.reshape(1, 128),
        W2, b2.reshape(1, 64), W3, b3.reshape(1, 32),
        W4.reshape(1, DG), b4.reshape(1, 1),
    )
    return out
